# Initial kernel scaffold; baseline (speedup 1.0000x reference)
import jax
import jax.numpy as jnp
from jax.experimental import pallas as pl


def kernel(data, angles):
    raise NotImplementedError("write your pallas kernel here")


# speedup vs baseline: 19.8417x; 19.8417x over previous
import jax
import jax.numpy as jnp
from jax.experimental import pallas as pl

_IN_CH = 128
_OUT_CH = 64
_STEPS = 5
_PTS = 16


def _aeg_body(data_ref, ang_ref, out_ref):
    a = ang_ref[...]
    ctab = jnp.cos(a) / _STEPS
    stab = jnp.sin(a) / _STEPS
    x = data_ref[...]
    for ix in range(_STEPS):
        z = (1.0 + x) * (_PTS / 2.0)
        posf = jnp.clip(jnp.round(z), 0.0, float(_PTS - 1))
        c = jnp.zeros_like(x)
        s = jnp.zeros_like(x)
        for k in range(_PTS):
            m = posf == float(k)
            c = jnp.where(m, ctab[ix, k], c)
            s = jnp.where(m, stab[ix, k], s)
        x = x + (c + x * s)
    r = jnp.sum(x, axis=1, keepdims=True)
    out_ref[...] = jnp.broadcast_to(r, (x.shape[0], _OUT_CH))


def kernel(data, angles):
    b = data.shape[0]
    return pl.pallas_call(
        _aeg_body,
        out_shape=jax.ShapeDtypeStruct((b, _OUT_CH), data.dtype),
    )(data, angles)


# speedup vs baseline: 22.9124x; 1.1548x over previous
import jax
import jax.numpy as jnp
from jax.experimental import pallas as pl

_IN_CH = 128
_OUT_CH = 64
_STEPS = 5
_PTS = 16


def _aeg_body(data_ref, ang_ref, out_ref):
    a = ang_ref[...]
    ctab = jnp.cos(a) / _STEPS
    stab = jnp.sin(a) / _STEPS
    x = data_ref[...]
    for ix in range(_STEPS):
        z = (1.0 + x) * (_PTS / 2.0)
        posf = jnp.clip(jnp.round(z), 0.0, float(_PTS - 1))
        pos = posf.astype(jnp.int32)
        cb = jnp.broadcast_to(ctab[ix][None, :], (x.shape[0], _PTS))
        sb = jnp.broadcast_to(stab[ix][None, :], (x.shape[0], _PTS))
        c = jnp.take_along_axis(cb, pos, axis=1)
        s = jnp.take_along_axis(sb, pos, axis=1)
        x = x + (c + x * s)
    r = jnp.sum(x, axis=1, keepdims=True)
    out_ref[...] = jnp.broadcast_to(r, (x.shape[0], _OUT_CH))


def kernel(data, angles):
    b = data.shape[0]
    return pl.pallas_call(
        _aeg_body,
        out_shape=jax.ShapeDtypeStruct((b, _OUT_CH), data.dtype),
    )(data, angles)
